# Initial kernel scaffold; baseline (speedup 1.0000x reference)
import jax
import jax.numpy as jnp
from jax.experimental import pallas as pl


def kernel(x, edge_index, W1, b1, W2, b2):
    raise NotImplementedError("write your pallas kernel here")


# speedup vs baseline: 11.0182x; 11.0182x over previous
import functools

import jax
import jax.numpy as jnp
from jax import lax
from jax.experimental import pallas as pl
from jax.experimental.pallas import tpu as pltpu
from jax.experimental.pallas import tpu_sc as plsc

_N = 10000
_E = 320000
_D = 128
_NC = 2
_NS = 16
_NW = _NC * _NS
_CH = 80
_EPW = _E // _NW
_NIT = _EPW // _CH
_NPAD = 10240
_RPT = _NPAD // _NS
_ZR = 128
_DPAD = 10240
_DPT = _DPAD // _NS

_L16 = 16


def _vs_mesh():
    return plsc.VectorSubcoreMesh(core_axis_name="c", subcore_axis_name="s")


def _sc_deg_body(dst_hbm, out_hbm, dst_v, ones_v, zero_v, acc_sh):
    c = lax.axis_index("c")
    s = lax.axis_index("s")
    wid = s * _NC + c

    for j in range(_CH // _L16):
        ones_v[pl.ds(j * _L16, _L16)] = jnp.ones((_L16,), jnp.float32)

    def zfill(j, carry):
        zero_v[pl.ds(j * _L16, _L16)] = jnp.zeros((_L16,), jnp.float32)
        return carry

    lax.fori_loop(0, _DPT // _L16, zfill, 0)
    pltpu.sync_copy(zero_v, acc_sh.at[pl.ds(s * _DPT, _DPT)])
    plsc.subcore_barrier()

    def body(i, carry):
        base = wid * _EPW + i * _CH
        pltpu.sync_copy(dst_hbm.at[pl.ds(base, _CH)], dst_v)
        pltpu.sync_copy(ones_v, acc_sh.at[dst_v], add=True)
        return carry

    lax.fori_loop(0, _NIT, body, 0)
    plsc.subcore_barrier()
    pltpu.sync_copy(acc_sh.at[pl.ds(s * _DPT, _DPT)],
                    out_hbm.at[c, pl.ds(s * _DPT, _DPT)])


@jax.jit
def _sc_deg(dst):
    return pl.kernel(
        _sc_deg_body,
        out_type=jax.ShapeDtypeStruct((_NC, _DPAD), jnp.float32),
        mesh=_vs_mesh(),
        scratch_types=[
            pltpu.VMEM((_CH,), jnp.int32),
            pltpu.VMEM((_CH,), jnp.float32),
            pltpu.VMEM((_DPT,), jnp.float32),
            pltpu.VMEM_SHARED((_DPAD,), jnp.float32),
        ],
    )(dst)


def _sc_gs_body(hs_hbm, src_hbm, dst_hbm, out_hbm,
                src_v, dst_v, rows_v, zero_v, acc_sh, sem):
    c = lax.axis_index("c")
    s = lax.axis_index("s")
    wid = s * _NC + c

    def zrow(i, carry):
        def zcol(j, carry2):
            zero_v[i, pl.ds(j * _L16, _L16)] = jnp.zeros((_L16,), jnp.float32)
            return carry2
        return lax.fori_loop(0, _D // _L16, zcol, carry)

    lax.fori_loop(0, _ZR, zrow, 0)
    for k in range(_RPT // _ZR):
        pltpu.sync_copy(zero_v, acc_sh.at[pl.ds(s * _RPT + k * _ZR, _ZR)])
    plsc.subcore_barrier()

    def body(i, carry):
        base = wid * _EPW + i * _CH
        pltpu.sync_copy(src_hbm.at[pl.ds(base, _CH)], src_v)
        pltpu.sync_copy(dst_hbm.at[pl.ds(base, _CH)], dst_v)
        pltpu.async_copy(hs_hbm.at[src_v], rows_v, sem).wait()
        pltpu.sync_copy(rows_v, acc_sh.at[dst_v], add=True)
        return carry

    lax.fori_loop(0, _NIT, body, 0)
    plsc.subcore_barrier()
    pltpu.sync_copy(acc_sh.at[pl.ds(s * _RPT, _RPT)],
                    out_hbm.at[c, pl.ds(s * _RPT, _RPT)])


@jax.jit
def _sc_gs(hs, src, dst):
    return pl.kernel(
        _sc_gs_body,
        out_type=jax.ShapeDtypeStruct((_NC, _NPAD, _D), jnp.float32),
        mesh=_vs_mesh(),
        scratch_types=[
            pltpu.VMEM((_CH,), jnp.int32),
            pltpu.VMEM((_CH,), jnp.int32),
            pltpu.VMEM((_CH, _D), jnp.float32),
            pltpu.VMEM((_ZR, _D), jnp.float32),
            pltpu.VMEM_SHARED((_NPAD, _D), jnp.float32),
            pltpu.SemaphoreType.DMA,
        ],
    )(hs, src, dst)


_BLK = 1000
_GRID = _N // _BLK


def _tc1_body(deg_ref, x_ref, w_ref, dis_ref, hs_ref):
    deg = deg_ref[0] + deg_ref[1]
    dis = jnp.where(deg > 0, lax.rsqrt(jnp.maximum(deg, 1.0)), 0.0)
    dis_ref[...] = dis
    hs_ref[...] = jnp.dot(x_ref[...], w_ref[...],
                          preferred_element_type=jnp.float32) * dis


@jax.jit
def _tc1(deg3, x, w1):
    return pl.pallas_call(
        _tc1_body,
        grid=(_GRID,),
        in_specs=[
            pl.BlockSpec((_NC, _BLK, 1), lambda i: (0, i, 0)),
            pl.BlockSpec((_BLK, _D), lambda i: (i, 0)),
            pl.BlockSpec((_D, _D), lambda i: (0, 0)),
        ],
        out_specs=[
            pl.BlockSpec((_BLK, 1), lambda i: (i, 0)),
            pl.BlockSpec((_BLK, _D), lambda i: (i, 0)),
        ],
        out_shape=[
            jax.ShapeDtypeStruct((_N, 1), jnp.float32),
            jax.ShapeDtypeStruct((_N, _D), jnp.float32),
        ],
    )(deg3, x, w1)


def _tc2_body(agg_ref, dis_ref, b_ref, w_ref, hs_ref):
    dis = dis_ref[...]
    h = jnp.maximum(dis * (agg_ref[0] + agg_ref[1]) + b_ref[...], 0.0)
    hs_ref[...] = jnp.dot(h, w_ref[...],
                          preferred_element_type=jnp.float32) * dis


@jax.jit
def _tc2(agg, dis, b1, w2):
    return pl.pallas_call(
        _tc2_body,
        grid=(_GRID,),
        in_specs=[
            pl.BlockSpec((_NC, _BLK, _D), lambda i: (0, i, 0)),
            pl.BlockSpec((_BLK, 1), lambda i: (i, 0)),
            pl.BlockSpec((1, _D), lambda i: (0, 0)),
            pl.BlockSpec((_D, _D), lambda i: (0, 0)),
        ],
        out_specs=pl.BlockSpec((_BLK, _D), lambda i: (i, 0)),
        out_shape=jax.ShapeDtypeStruct((_N, _D), jnp.float32),
    )(agg, dis, b1, w2)


def _tc3_body(agg_ref, dis_ref, b_ref, out_ref):
    out_ref[...] = dis_ref[...] * (agg_ref[0] + agg_ref[1]) + b_ref[...]


@jax.jit
def _tc3(agg, dis, b2):
    return pl.pallas_call(
        _tc3_body,
        grid=(_GRID,),
        in_specs=[
            pl.BlockSpec((_NC, _BLK, _D), lambda i: (0, i, 0)),
            pl.BlockSpec((_BLK, 1), lambda i: (i, 0)),
            pl.BlockSpec((1, _D), lambda i: (0, 0)),
        ],
        out_specs=pl.BlockSpec((_BLK, _D), lambda i: (i, 0)),
        out_shape=jax.ShapeDtypeStruct((_N, _D), jnp.float32),
    )(agg, dis, b2)


def kernel(x, edge_index, W1, b1, W2, b2):
    src = edge_index[0]
    dst = edge_index[1]
    deg_parts = _sc_deg(dst)
    deg3 = deg_parts.reshape(_NC, _DPAD, 1)
    dis, hs1 = _tc1(deg3, x, W1)
    agg1 = _sc_gs(hs1, src, dst)
    hs2 = _tc2(agg1, dis, b1.reshape(1, _D), W2)
    agg2 = _sc_gs(hs2, src, dst)
    return _tc3(agg2, dis, b2.reshape(1, _D))


# speedup vs baseline: 26.6902x; 2.4224x over previous
import functools

import jax
import jax.numpy as jnp
from jax import lax
from jax.experimental import pallas as pl
from jax.experimental.pallas import tpu as pltpu
from jax.experimental.pallas import tpu_sc as plsc

_N = 10000
_E = 320000
_D = 128
_NC = 2
_NS = 16
_NW = _NC * _NS
_CH = 80
_EPW = _E // _NW
_NIT = _EPW // _CH
_NPAD = 10240
_RPT = _NPAD // _NS
_ZR = 64
_DPAD = 10240
_DPT = _DPAD // _NS

_L16 = 16


def _vs_mesh():
    return plsc.VectorSubcoreMesh(core_axis_name="c", subcore_axis_name="s")


_DGRP = 5


def _sc_deg_body(dst_hbm, out_hbm, dst_all, ones_v, zero_v, acc_sh, isem, dsem):
    c = lax.axis_index("c")
    s = lax.axis_index("s")
    wid = s * _NC + c

    idx_d = pltpu.async_copy(dst_hbm.at[wid], dst_all, isem)

    for j in range(_CH // _L16):
        ones_v[pl.ds(j * _L16, _L16)] = jnp.ones((_L16,), jnp.float32)

    def zfill(j, carry):
        zero_v[pl.ds(j * _L16, _L16)] = jnp.zeros((_L16,), jnp.float32)
        return carry

    lax.fori_loop(0, _DPT // _L16, zfill, 0)
    pltpu.sync_copy(zero_v, acc_sh.at[pl.ds(s * _DPT, _DPT)])
    idx_d.wait()
    plsc.subcore_barrier()

    def body(j, carry):
        for b in range(_DGRP):
            pltpu.async_copy(ones_v, acc_sh.at[dst_all.at[j * _DGRP + b]],
                             dsem, add=True)
        for b in range(_DGRP):
            pltpu.make_async_copy(ones_v, acc_sh.at[dst_all.at[j * _DGRP + b]],
                                  dsem).wait()
        return carry

    lax.fori_loop(0, _NIT // _DGRP, body, 0)
    plsc.subcore_barrier()
    pltpu.sync_copy(acc_sh.at[pl.ds(s * _DPT, _DPT)],
                    out_hbm.at[c, pl.ds(s * _DPT, _DPT)])


@jax.jit
def _sc_deg(dst3):
    return pl.kernel(
        _sc_deg_body,
        out_type=jax.ShapeDtypeStruct((_NC, _DPAD), jnp.float32),
        mesh=_vs_mesh(),
        scratch_types=[
            pltpu.VMEM((_NIT, _CH), jnp.int32),
            pltpu.VMEM((_CH,), jnp.float32),
            pltpu.VMEM((_DPT,), jnp.float32),
            pltpu.VMEM_SHARED((_DPAD,), jnp.float32),
            pltpu.SemaphoreType.DMA,
            pltpu.SemaphoreType.DMA,
        ],
    )(dst3)


def _sc_gs_body(hs_hbm, src_hbm, dst_hbm, out_hbm,
                src_all, dst_all, rows0, rows1, acc_sh,
                isem, gsem0, gsem1):
    c = lax.axis_index("c")
    s = lax.axis_index("s")
    wid = s * _NC + c

    idx_s = pltpu.async_copy(src_hbm.at[pl.ds(wid * _EPW, _EPW)], src_all, isem)
    idx_d = pltpu.async_copy(dst_hbm.at[wid], dst_all, isem)

    def zrow(i, carry):
        def zcol(j, carry2):
            rows0[i, pl.ds(j * _L16, _L16)] = jnp.zeros((_L16,), jnp.float32)
            return carry2
        return lax.fori_loop(0, _D // _L16, zcol, carry)

    lax.fori_loop(0, _CH, zrow, 0)
    for k in range(_RPT // _CH):
        pltpu.sync_copy(rows0, acc_sh.at[pl.ds(s * _RPT + k * _CH, _CH)])
    idx_s.wait()
    idx_d.wait()
    plsc.subcore_barrier()

    pltpu.async_copy(hs_hbm.at[src_all.at[pl.ds(0, _CH)]], rows0, gsem0)

    def body(j, carry):
        i0 = j * 2
        i1 = i0 + 1
        pltpu.async_copy(hs_hbm.at[src_all.at[pl.ds(i1 * _CH, _CH)]],
                         rows1, gsem1)
        pltpu.make_async_copy(hs_hbm.at[src_all.at[pl.ds(i0 * _CH, _CH)]],
                              rows0, gsem0).wait()
        pltpu.sync_copy(rows0, acc_sh.at[dst_all.at[i0]], add=True)
        pltpu.async_copy(hs_hbm.at[src_all.at[pl.ds((i0 + 2) * _CH, _CH)]],
                         rows0, gsem0)
        pltpu.make_async_copy(hs_hbm.at[src_all.at[pl.ds(i1 * _CH, _CH)]],
                              rows1, gsem1).wait()
        pltpu.sync_copy(rows1, acc_sh.at[dst_all.at[i1]], add=True)
        return carry

    lax.fori_loop(0, (_NIT - 1) // 2, body, 0)
    pltpu.make_async_copy(hs_hbm.at[src_all.at[pl.ds((_NIT - 1) * _CH, _CH)]],
                          rows0, gsem0).wait()
    pltpu.sync_copy(rows0, acc_sh.at[dst_all.at[_NIT - 1]], add=True)

    plsc.subcore_barrier()
    pltpu.sync_copy(acc_sh.at[pl.ds(s * _RPT, _RPT)],
                    out_hbm.at[c, pl.ds(s * _RPT, _RPT)])


@jax.jit
def _sc_gs(hs, src3, dst3):
    return pl.kernel(
        _sc_gs_body,
        out_type=jax.ShapeDtypeStruct((_NC, _NPAD, _D), jnp.float32),
        mesh=_vs_mesh(),
        scratch_types=[
            pltpu.VMEM((_EPW,), jnp.int32),
            pltpu.VMEM((_NIT, _CH), jnp.int32),
            pltpu.VMEM((_CH, _D), jnp.float32),
            pltpu.VMEM((_CH, _D), jnp.float32),
            pltpu.VMEM_SHARED((_NPAD, _D), jnp.float32),
            pltpu.SemaphoreType.DMA,
            pltpu.SemaphoreType.DMA,
            pltpu.SemaphoreType.DMA,
        ],
    )(hs, src3, dst3)


_BLK = 1000
_GRID = _N // _BLK


def _tc1_body(deg_ref, x_ref, w_ref, dis_ref, hs_ref):
    deg = deg_ref[0] + deg_ref[1]
    dis = jnp.where(deg > 0, lax.rsqrt(jnp.maximum(deg, 1.0)), 0.0)
    dis_ref[...] = dis
    hs_ref[...] = jnp.dot(x_ref[...], w_ref[...],
                          preferred_element_type=jnp.float32) * dis


@jax.jit
def _tc1(deg3, x, w1):
    return pl.pallas_call(
        _tc1_body,
        grid=(_GRID,),
        in_specs=[
            pl.BlockSpec((_NC, _BLK, 1), lambda i: (0, i, 0)),
            pl.BlockSpec((_BLK, _D), lambda i: (i, 0)),
            pl.BlockSpec((_D, _D), lambda i: (0, 0)),
        ],
        out_specs=[
            pl.BlockSpec((_BLK, 1), lambda i: (i, 0)),
            pl.BlockSpec((_BLK, _D), lambda i: (i, 0)),
        ],
        out_shape=[
            jax.ShapeDtypeStruct((_N, 1), jnp.float32),
            jax.ShapeDtypeStruct((_N, _D), jnp.float32),
        ],
    )(deg3, x, w1)


def _tc2_body(agg_ref, dis_ref, b_ref, w_ref, hs_ref):
    dis = dis_ref[...]
    h = jnp.maximum(dis * (agg_ref[0] + agg_ref[1]) + b_ref[...], 0.0)
    hs_ref[...] = jnp.dot(h, w_ref[...],
                          preferred_element_type=jnp.float32) * dis


@jax.jit
def _tc2(agg, dis, b1, w2):
    return pl.pallas_call(
        _tc2_body,
        grid=(_GRID,),
        in_specs=[
            pl.BlockSpec((_NC, _BLK, _D), lambda i: (0, i, 0)),
            pl.BlockSpec((_BLK, 1), lambda i: (i, 0)),
            pl.BlockSpec((1, _D), lambda i: (0, 0)),
            pl.BlockSpec((_D, _D), lambda i: (0, 0)),
        ],
        out_specs=pl.BlockSpec((_BLK, _D), lambda i: (i, 0)),
        out_shape=jax.ShapeDtypeStruct((_N, _D), jnp.float32),
    )(agg, dis, b1, w2)


def _tc3_body(agg_ref, dis_ref, b_ref, out_ref):
    out_ref[...] = dis_ref[...] * (agg_ref[0] + agg_ref[1]) + b_ref[...]


@jax.jit
def _tc3(agg, dis, b2):
    return pl.pallas_call(
        _tc3_body,
        grid=(_GRID,),
        in_specs=[
            pl.BlockSpec((_NC, _BLK, _D), lambda i: (0, i, 0)),
            pl.BlockSpec((_BLK, 1), lambda i: (i, 0)),
            pl.BlockSpec((1, _D), lambda i: (0, 0)),
        ],
        out_specs=pl.BlockSpec((_BLK, _D), lambda i: (i, 0)),
        out_shape=jax.ShapeDtypeStruct((_N, _D), jnp.float32),
    )(agg, dis, b2)


def kernel(x, edge_index, W1, b1, W2, b2):
    src = edge_index[0]
    dst3 = edge_index[1].reshape(_NW, _NIT, _CH)
    deg_parts = _sc_deg(dst3)
    deg3 = deg_parts.reshape(_NC, _DPAD, 1)
    dis, hs1 = _tc1(deg3, x, W1)
    agg1 = _sc_gs(hs1, src, dst3)
    hs2 = _tc2(agg1, dis, b1.reshape(1, _D), W2)
    agg2 = _sc_gs(hs2, src, dst3)
    return _tc3(agg2, dis, b2.reshape(1, _D))


# speedup vs baseline: 28.8390x; 1.0805x over previous
import functools

import jax
import jax.numpy as jnp
from jax import lax
from jax.experimental import pallas as pl
from jax.experimental.pallas import tpu as pltpu
from jax.experimental.pallas import tpu_sc as plsc

_N = 10000
_E = 320000
_D = 128
_NC = 2
_NS = 16
_NW = _NC * _NS
_CH = 80
_EPW = _E // _NW
_NIT = _EPW // _CH
_GCH = 128
_GNF = _EPW // _GCH
_GTL = _EPW - _GNF * _GCH
_NPAD = 10240
_RPT = _NPAD // _NS
_ZR = 64
_DPAD = 10240
_DPT = _DPAD // _NS

_L16 = 16


def _vs_mesh():
    return plsc.VectorSubcoreMesh(core_axis_name="c", subcore_axis_name="s")


_DGRP = 5


def _sc_deg_body(dst_hbm, out_hbm, dst_all, ones_v, zero_v, acc_sh, isem, dsem):
    c = lax.axis_index("c")
    s = lax.axis_index("s")
    wid = s * _NC + c

    idx_d = pltpu.async_copy(dst_hbm.at[wid], dst_all, isem)

    for j in range(_CH // _L16):
        ones_v[pl.ds(j * _L16, _L16)] = jnp.ones((_L16,), jnp.float32)

    def zfill(j, carry):
        zero_v[pl.ds(j * _L16, _L16)] = jnp.zeros((_L16,), jnp.float32)
        return carry

    lax.fori_loop(0, _DPT // _L16, zfill, 0)
    pltpu.sync_copy(zero_v, acc_sh.at[pl.ds(s * _DPT, _DPT)])
    idx_d.wait()
    plsc.subcore_barrier()

    def body(j, carry):
        for b in range(_DGRP):
            pltpu.async_copy(ones_v, acc_sh.at[dst_all.at[j * _DGRP + b]],
                             dsem, add=True)
        for b in range(_DGRP):
            pltpu.make_async_copy(ones_v, acc_sh.at[dst_all.at[j * _DGRP + b]],
                                  dsem).wait()
        return carry

    lax.fori_loop(0, _NIT // _DGRP, body, 0)
    plsc.subcore_barrier()
    pltpu.sync_copy(acc_sh.at[pl.ds(s * _DPT, _DPT)],
                    out_hbm.at[c, pl.ds(s * _DPT, _DPT)])


@jax.jit
def _sc_deg(dst3):
    return pl.kernel(
        _sc_deg_body,
        out_type=jax.ShapeDtypeStruct((_NC, _DPAD), jnp.float32),
        mesh=_vs_mesh(),
        scratch_types=[
            pltpu.VMEM((_NIT, _CH), jnp.int32),
            pltpu.VMEM((_CH,), jnp.float32),
            pltpu.VMEM((_DPT,), jnp.float32),
            pltpu.VMEM_SHARED((_DPAD,), jnp.float32),
            pltpu.SemaphoreType.DMA,
            pltpu.SemaphoreType.DMA,
        ],
    )(dst3)


def _sc_gs_body(hs_hbm, src_hbm, dst_hbm, out_hbm,
                src_all, d0, d1, rows0, rows1, acc_sh,
                isem, gsem0, gsem1, dsem0, dsem1):
    c = lax.axis_index("c")
    s = lax.axis_index("s")
    wid = s * _NC + c
    ebase = wid * _EPW

    idx_s = pltpu.async_copy(src_hbm.at[pl.ds(ebase, _EPW)], src_all, isem)

    def zrow(i, carry):
        def zcol(j, carry2):
            rows0[i, pl.ds(j * _L16, _L16)] = jnp.zeros((_L16,), jnp.float32)
            return carry2
        return lax.fori_loop(0, _D // _L16, zcol, carry)

    lax.fori_loop(0, _GCH, zrow, 0)
    for k in range(_RPT // _GCH):
        pltpu.sync_copy(rows0, acc_sh.at[pl.ds(s * _RPT + k * _GCH, _GCH)])
    idx_s.wait()
    plsc.subcore_barrier()

    pltpu.async_copy(hs_hbm.at[src_all.at[pl.ds(0, _GCH)]], rows0, gsem0)
    pltpu.async_copy(dst_hbm.at[pl.ds(ebase, _GCH)], d0, dsem0)

    def body(j, carry):
        i0 = j * 2
        i1 = i0 + 1
        pltpu.async_copy(hs_hbm.at[src_all.at[pl.ds(i1 * _GCH, _GCH)]],
                         rows1, gsem1)
        pltpu.async_copy(dst_hbm.at[pl.ds(ebase + i1 * _GCH, _GCH)], d1, dsem1)
        pltpu.make_async_copy(hs_hbm.at[src_all.at[pl.ds(i0 * _GCH, _GCH)]],
                              rows0, gsem0).wait()
        pltpu.make_async_copy(dst_hbm.at[pl.ds(ebase + i0 * _GCH, _GCH)],
                              d0, dsem0).wait()
        pltpu.sync_copy(rows0, acc_sh.at[d0], add=True)
        pltpu.async_copy(hs_hbm.at[src_all.at[pl.ds((i0 + 2) * _GCH, _GCH)]],
                         rows0, gsem0)
        pltpu.async_copy(dst_hbm.at[pl.ds(ebase + (i0 + 2) * _GCH, _GCH)],
                         d0, dsem0)
        pltpu.make_async_copy(hs_hbm.at[src_all.at[pl.ds(i1 * _GCH, _GCH)]],
                              rows1, gsem1).wait()
        pltpu.make_async_copy(dst_hbm.at[pl.ds(ebase + i1 * _GCH, _GCH)],
                              d1, dsem1).wait()
        pltpu.sync_copy(rows1, acc_sh.at[d1], add=True)
        return carry

    lax.fori_loop(0, _GNF // 2 - 1, body, 0)
    pltpu.make_async_copy(hs_hbm.at[src_all.at[pl.ds((_GNF - 2) * _GCH, _GCH)]],
                          rows0, gsem0).wait()
    pltpu.make_async_copy(dst_hbm.at[pl.ds(ebase + (_GNF - 2) * _GCH, _GCH)],
                          d0, dsem0).wait()
    pltpu.async_copy(hs_hbm.at[src_all.at[pl.ds((_GNF - 1) * _GCH, _GCH)]],
                     rows1, gsem1)
    pltpu.async_copy(dst_hbm.at[pl.ds(ebase + (_GNF - 1) * _GCH, _GCH)],
                     d1, dsem1)
    pltpu.sync_copy(rows0, acc_sh.at[d0], add=True)
    pltpu.make_async_copy(hs_hbm.at[src_all.at[pl.ds((_GNF - 1) * _GCH, _GCH)]],
                          rows1, gsem1).wait()
    pltpu.make_async_copy(dst_hbm.at[pl.ds(ebase + (_GNF - 1) * _GCH, _GCH)],
                          d1, dsem1).wait()
    pltpu.async_copy(hs_hbm.at[src_all.at[pl.ds(_GNF * _GCH, _GTL)]],
                     rows0.at[pl.ds(0, _GTL)], gsem0)
    pltpu.async_copy(dst_hbm.at[pl.ds(ebase + _GNF * _GCH, _GTL)],
                     d0.at[pl.ds(0, _GTL)], dsem0)
    pltpu.sync_copy(rows1, acc_sh.at[d1], add=True)
    pltpu.make_async_copy(hs_hbm.at[src_all.at[pl.ds(_GNF * _GCH, _GTL)]],
                          rows0.at[pl.ds(0, _GTL)], gsem0).wait()
    pltpu.make_async_copy(dst_hbm.at[pl.ds(ebase + _GNF * _GCH, _GTL)],
                          d0.at[pl.ds(0, _GTL)], dsem0).wait()
    pltpu.sync_copy(rows0.at[pl.ds(0, _GTL)],
                    acc_sh.at[d0.at[pl.ds(0, _GTL)]], add=True)

    plsc.subcore_barrier()
    pltpu.sync_copy(acc_sh.at[pl.ds(s * _RPT, _RPT)],
                    out_hbm.at[c, pl.ds(s * _RPT, _RPT)])


@jax.jit
def _sc_gs(hs, src3, dst3):
    return pl.kernel(
        _sc_gs_body,
        out_type=jax.ShapeDtypeStruct((_NC, _NPAD, _D), jnp.float32),
        mesh=_vs_mesh(),
        scratch_types=[
            pltpu.VMEM((_EPW,), jnp.int32),
            pltpu.VMEM((_GCH,), jnp.int32),
            pltpu.VMEM((_GCH,), jnp.int32),
            pltpu.VMEM((_GCH, _D), jnp.float32),
            pltpu.VMEM((_GCH, _D), jnp.float32),
            pltpu.VMEM_SHARED((_NPAD, _D), jnp.float32),
            pltpu.SemaphoreType.DMA,
            pltpu.SemaphoreType.DMA,
            pltpu.SemaphoreType.DMA,
            pltpu.SemaphoreType.DMA,
            pltpu.SemaphoreType.DMA,
        ],
    )(hs, src3, dst3)


_BLK = 1000
_GRID = _N // _BLK


def _tc1_body(deg_ref, x_ref, w_ref, dis_ref, hs_ref):
    deg = deg_ref[0] + deg_ref[1]
    dis = jnp.where(deg > 0, lax.rsqrt(jnp.maximum(deg, 1.0)), 0.0)
    dis_ref[...] = dis
    hs_ref[...] = jnp.dot(x_ref[...], w_ref[...],
                          preferred_element_type=jnp.float32) * dis


@jax.jit
def _tc1(deg3, x, w1):
    return pl.pallas_call(
        _tc1_body,
        grid=(_GRID,),
        in_specs=[
            pl.BlockSpec((_NC, _BLK, 1), lambda i: (0, i, 0)),
            pl.BlockSpec((_BLK, _D), lambda i: (i, 0)),
            pl.BlockSpec((_D, _D), lambda i: (0, 0)),
        ],
        out_specs=[
            pl.BlockSpec((_BLK, 1), lambda i: (i, 0)),
            pl.BlockSpec((_BLK, _D), lambda i: (i, 0)),
        ],
        out_shape=[
            jax.ShapeDtypeStruct((_N, 1), jnp.float32),
            jax.ShapeDtypeStruct((_N, _D), jnp.float32),
        ],
    )(deg3, x, w1)


def _tc2_body(agg_ref, dis_ref, b_ref, w_ref, hs_ref):
    dis = dis_ref[...]
    h = jnp.maximum(dis * (agg_ref[0] + agg_ref[1]) + b_ref[...], 0.0)
    hs_ref[...] = jnp.dot(h, w_ref[...],
                          preferred_element_type=jnp.float32) * dis


@jax.jit
def _tc2(agg, dis, b1, w2):
    return pl.pallas_call(
        _tc2_body,
        grid=(_GRID,),
        in_specs=[
            pl.BlockSpec((_NC, _BLK, _D), lambda i: (0, i, 0)),
            pl.BlockSpec((_BLK, 1), lambda i: (i, 0)),
            pl.BlockSpec((1, _D), lambda i: (0, 0)),
            pl.BlockSpec((_D, _D), lambda i: (0, 0)),
        ],
        out_specs=pl.BlockSpec((_BLK, _D), lambda i: (i, 0)),
        out_shape=jax.ShapeDtypeStruct((_N, _D), jnp.float32),
    )(agg, dis, b1, w2)


def _tc3_body(agg_ref, dis_ref, b_ref, out_ref):
    out_ref[...] = dis_ref[...] * (agg_ref[0] + agg_ref[1]) + b_ref[...]


@jax.jit
def _tc3(agg, dis, b2):
    return pl.pallas_call(
        _tc3_body,
        grid=(_GRID,),
        in_specs=[
            pl.BlockSpec((_NC, _BLK, _D), lambda i: (0, i, 0)),
            pl.BlockSpec((_BLK, 1), lambda i: (i, 0)),
            pl.BlockSpec((1, _D), lambda i: (0, 0)),
        ],
        out_specs=pl.BlockSpec((_BLK, _D), lambda i: (i, 0)),
        out_shape=jax.ShapeDtypeStruct((_N, _D), jnp.float32),
    )(agg, dis, b2)


def kernel(x, edge_index, W1, b1, W2, b2):
    src = edge_index[0]
    dst = edge_index[1]
    dst3 = dst.reshape(_NW, _NIT, _CH)
    deg_parts = _sc_deg(dst3)
    deg3 = deg_parts.reshape(_NC, _DPAD, 1)
    dis, hs1 = _tc1(deg3, x, W1)
    agg1 = _sc_gs(hs1, src, dst)
    hs2 = _tc2(agg1, dis, b1.reshape(1, _D), W2)
    agg2 = _sc_gs(hs2, src, dst)
    return _tc3(agg2, dis, b2.reshape(1, _D))


# speedup vs baseline: 31.0934x; 1.0782x over previous
import functools

import jax
import jax.numpy as jnp
from jax import lax
from jax.experimental import pallas as pl
from jax.experimental.pallas import tpu as pltpu
from jax.experimental.pallas import tpu_sc as plsc

_N = 10000
_E = 320000
_D = 128
_NC = 2
_NS = 16
_NW = _NC * _NS
_CH = 80
_EPW = _E // _NW
_NIT = _EPW // _CH
_GCH = 80
_GN = _EPW // _GCH
_GK = 4
_NPAD = 10240
_RPT = _NPAD // _NS
_ZR = 64
_DPAD = 10240
_DPT = _DPAD // _NS

_L16 = 16


def _vs_mesh():
    return plsc.VectorSubcoreMesh(core_axis_name="c", subcore_axis_name="s")


_DGRP = 5


def _sc_deg_body(dst_hbm, out_hbm, dst_all, ones_v, zero_v, acc_sh, isem, dsem):
    c = lax.axis_index("c")
    s = lax.axis_index("s")
    wid = s * _NC + c

    idx_d = pltpu.async_copy(dst_hbm.at[wid], dst_all, isem)

    for j in range(_CH // _L16):
        ones_v[pl.ds(j * _L16, _L16)] = jnp.ones((_L16,), jnp.float32)

    def zfill(j, carry):
        zero_v[pl.ds(j * _L16, _L16)] = jnp.zeros((_L16,), jnp.float32)
        return carry

    lax.fori_loop(0, _DPT // _L16, zfill, 0)
    pltpu.sync_copy(zero_v, acc_sh.at[pl.ds(s * _DPT, _DPT)])
    idx_d.wait()
    plsc.subcore_barrier()

    def body(j, carry):
        for b in range(_DGRP):
            pltpu.async_copy(ones_v, acc_sh.at[dst_all.at[j * _DGRP + b]],
                             dsem, add=True)
        for b in range(_DGRP):
            pltpu.make_async_copy(ones_v, acc_sh.at[dst_all.at[j * _DGRP + b]],
                                  dsem).wait()
        return carry

    lax.fori_loop(0, _NIT // _DGRP, body, 0)
    plsc.subcore_barrier()
    pltpu.sync_copy(acc_sh.at[pl.ds(s * _DPT, _DPT)],
                    out_hbm.at[c, pl.ds(s * _DPT, _DPT)])


@jax.jit
def _sc_deg(dst3):
    return pl.kernel(
        _sc_deg_body,
        out_type=jax.ShapeDtypeStruct((_NC, _DPAD), jnp.float32),
        mesh=_vs_mesh(),
        scratch_types=[
            pltpu.VMEM((_NIT, _CH), jnp.int32),
            pltpu.VMEM((_CH,), jnp.float32),
            pltpu.VMEM((_DPT,), jnp.float32),
            pltpu.VMEM_SHARED((_DPAD,), jnp.float32),
            pltpu.SemaphoreType.DMA,
            pltpu.SemaphoreType.DMA,
        ],
    )(dst3)


def _sc_gs_body(hs_hbm, src_hbm, dst_hbm, out_hbm, *scr):
    rows = scr[0:_GK]
    sidx = scr[_GK:2 * _GK]
    didx = scr[2 * _GK:3 * _GK]
    acc_sh = scr[3 * _GK]
    ssem = scr[3 * _GK + 1:4 * _GK + 1]
    dsem = scr[4 * _GK + 1:5 * _GK + 1]
    gsem = scr[5 * _GK + 1:6 * _GK + 1]

    c = lax.axis_index("c")
    s = lax.axis_index("s")
    wid = s * _NC + c
    ebase = wid * _EPW

    def p_idx(i, sl):
        pltpu.async_copy(src_hbm.at[pl.ds(ebase + i * _GCH, _GCH)],
                         sidx[sl], ssem[sl])
        pltpu.async_copy(dst_hbm.at[pl.ds(ebase + i * _GCH, _GCH)],
                         didx[sl], dsem[sl])

    def p_gather(i, sl):
        pltpu.make_async_copy(src_hbm.at[pl.ds(ebase + i * _GCH, _GCH)],
                              sidx[sl], ssem[sl]).wait()
        pltpu.async_copy(hs_hbm.at[sidx[sl].at[pl.ds(0, _GCH)]],
                         rows[sl], gsem[sl])

    def p_consume(i, sl):
        pltpu.make_async_copy(hs_hbm.at[sidx[sl].at[pl.ds(0, _GCH)]],
                              rows[sl], gsem[sl]).wait()
        pltpu.make_async_copy(dst_hbm.at[pl.ds(ebase + i * _GCH, _GCH)],
                              didx[sl], dsem[sl]).wait()
        pltpu.sync_copy(rows[sl], acc_sh.at[didx[sl].at[pl.ds(0, _GCH)]],
                        add=True)

    for i in range(_GK - 1):
        p_idx(i, i)

    def zrow(i, carry):
        def zcol(j, carry2):
            rows[0][i, pl.ds(j * _L16, _L16)] = jnp.zeros((_L16,), jnp.float32)
            return carry2
        return lax.fori_loop(0, _D // _L16, zcol, carry)

    lax.fori_loop(0, _GCH, zrow, 0)
    for k in range(_RPT // _GCH):
        pltpu.sync_copy(rows[0], acc_sh.at[pl.ds(s * _RPT + k * _GCH, _GCH)])
    plsc.subcore_barrier()

    p_gather(0, 0)
    p_gather(1, 1)

    def body(j, carry):
        i = j * _GK
        for t in range(_GK):
            p_idx(i + t + _GK - 1, (t + _GK - 1) % _GK)
            p_gather(i + t + 2, (t + 2) % _GK)
            p_consume(i + t, t)
        return carry

    lax.fori_loop(0, _GN // _GK - 1, body, 0)
    b = (_GN // _GK - 1) * _GK
    p_idx(b + 3, (b + 3) % _GK)
    p_gather(b + 2, (b + 2) % _GK)
    p_consume(b, b % _GK)
    p_idx(b + 4, (b + 4) % _GK)
    p_gather(b + 3, (b + 3) % _GK)
    p_consume(b + 1, (b + 1) % _GK)
    p_gather(b + 4, (b + 4) % _GK)
    p_consume(b + 2, (b + 2) % _GK)
    p_consume(b + 3, (b + 3) % _GK)
    p_consume(b + 4, (b + 4) % _GK)

    plsc.subcore_barrier()
    pltpu.sync_copy(acc_sh.at[pl.ds(s * _RPT, _RPT)],
                    out_hbm.at[c, pl.ds(s * _RPT, _RPT)])


@jax.jit
def _sc_gs(hs, src3, dst3):
    return pl.kernel(
        _sc_gs_body,
        out_type=jax.ShapeDtypeStruct((_NC, _NPAD, _D), jnp.float32),
        mesh=_vs_mesh(),
        scratch_types=(
            [pltpu.VMEM((_GCH, _D), jnp.float32) for _ in range(_GK)]
            + [pltpu.VMEM((_GCH,), jnp.int32) for _ in range(2 * _GK)]
            + [pltpu.VMEM_SHARED((_NPAD, _D), jnp.float32)]
            + [pltpu.SemaphoreType.DMA for _ in range(3 * _GK)]
        ),
    )(hs, src3, dst3)


_BLK = 1000
_GRID = _N // _BLK


def _tc1_body(deg_ref, x_ref, w_ref, dis_ref, hs_ref):
    deg = deg_ref[0] + deg_ref[1]
    dis = jnp.where(deg > 0, lax.rsqrt(jnp.maximum(deg, 1.0)), 0.0)
    dis_ref[...] = dis
    hs_ref[...] = jnp.dot(x_ref[...], w_ref[...],
                          preferred_element_type=jnp.float32) * dis


@jax.jit
def _tc1(deg3, x, w1):
    return pl.pallas_call(
        _tc1_body,
        grid=(_GRID,),
        in_specs=[
            pl.BlockSpec((_NC, _BLK, 1), lambda i: (0, i, 0)),
            pl.BlockSpec((_BLK, _D), lambda i: (i, 0)),
            pl.BlockSpec((_D, _D), lambda i: (0, 0)),
        ],
        out_specs=[
            pl.BlockSpec((_BLK, 1), lambda i: (i, 0)),
            pl.BlockSpec((_BLK, _D), lambda i: (i, 0)),
        ],
        out_shape=[
            jax.ShapeDtypeStruct((_N, 1), jnp.float32),
            jax.ShapeDtypeStruct((_N, _D), jnp.float32),
        ],
    )(deg3, x, w1)


def _tc2_body(agg_ref, dis_ref, b_ref, w_ref, hs_ref):
    dis = dis_ref[...]
    h = jnp.maximum(dis * (agg_ref[0] + agg_ref[1]) + b_ref[...], 0.0)
    hs_ref[...] = jnp.dot(h, w_ref[...],
                          preferred_element_type=jnp.float32) * dis


@jax.jit
def _tc2(agg, dis, b1, w2):
    return pl.pallas_call(
        _tc2_body,
        grid=(_GRID,),
        in_specs=[
            pl.BlockSpec((_NC, _BLK, _D), lambda i: (0, i, 0)),
            pl.BlockSpec((_BLK, 1), lambda i: (i, 0)),
            pl.BlockSpec((1, _D), lambda i: (0, 0)),
            pl.BlockSpec((_D, _D), lambda i: (0, 0)),
        ],
        out_specs=pl.BlockSpec((_BLK, _D), lambda i: (i, 0)),
        out_shape=jax.ShapeDtypeStruct((_N, _D), jnp.float32),
    )(agg, dis, b1, w2)


def _tc3_body(agg_ref, dis_ref, b_ref, out_ref):
    out_ref[...] = dis_ref[...] * (agg_ref[0] + agg_ref[1]) + b_ref[...]


@jax.jit
def _tc3(agg, dis, b2):
    return pl.pallas_call(
        _tc3_body,
        grid=(_GRID,),
        in_specs=[
            pl.BlockSpec((_NC, _BLK, _D), lambda i: (0, i, 0)),
            pl.BlockSpec((_BLK, 1), lambda i: (i, 0)),
            pl.BlockSpec((1, _D), lambda i: (0, 0)),
        ],
        out_specs=pl.BlockSpec((_BLK, _D), lambda i: (i, 0)),
        out_shape=jax.ShapeDtypeStruct((_N, _D), jnp.float32),
    )(agg, dis, b2)


def kernel(x, edge_index, W1, b1, W2, b2):
    src = edge_index[0]
    dst = edge_index[1]
    dst3 = dst.reshape(_NW, _NIT, _CH)
    deg_parts = _sc_deg(dst3)
    deg3 = deg_parts.reshape(_NC, _DPAD, 1)
    dis, hs1 = _tc1(deg3, x, W1)
    agg1 = _sc_gs(hs1, src, dst)
    hs2 = _tc2(agg1, dis, b1.reshape(1, _D), W2)
    agg2 = _sc_gs(hs2, src, dst)
    return _tc3(agg2, dis, b2.reshape(1, _D))
